# baseline (speedup 1.0000x reference)
import functools

import jax
import jax.numpy as jnp
from jax import lax
from jax.experimental import pallas as pl
from jax.experimental.pallas import tpu as pltpu
from jax.experimental.pallas import tpu_sc as plsc

_T, _D, _E, _K, _FF, _SFF = 2048, 2048, 8, 2, 768, 2048
_BT = 256
_NBLK = 24
_NPAD = _NBLK * _BT


def _router_body(x_ref, wr_ref, wsg_ref, slots_ref, w_ref, sgate_ref, meta_ref,
                 stid_ref):
    x = x_ref[...]
    logits = jnp.dot(x, wr_ref[...], preferred_element_type=jnp.float32)
    idx = lax.broadcasted_iota(jnp.int32, (_T, _E), 1)
    m1 = jnp.max(logits, axis=1, keepdims=True)
    a1 = jnp.min(jnp.where(logits == m1, idx, _E), axis=1, keepdims=True)
    masked = jnp.where(idx == a1, -jnp.inf, logits)
    m2 = jnp.max(masked, axis=1, keepdims=True)
    a2 = jnp.min(jnp.where(masked == m2, idx, _E), axis=1, keepdims=True)
    w1 = jax.nn.sigmoid(m1 - m2)
    w_ref[...] = jnp.concatenate([w1, 1.0 - w1], axis=1)

    sel = ((idx == a1) | (idx == a2)).astype(jnp.float32)

    r = lax.broadcasted_iota(jnp.int32, (_BT, _BT), 0)
    c = lax.broadcasted_iota(jnp.int32, (_BT, _BT), 1)
    tri = (c < r).astype(jnp.float32)
    nb = _T // _BT
    bsums = []
    ranks_local = []
    for b in range(nb):
        sb = sel[b * _BT:(b + 1) * _BT]
        ranks_local.append(jnp.dot(tri, sb, preferred_element_type=jnp.float32))
        bsums.append(jnp.sum(sb, axis=0, keepdims=True))
    bsum = jnp.concatenate(bsums, axis=0)
    r8 = lax.broadcasted_iota(jnp.int32, (nb, nb), 0)
    c8 = lax.broadcasted_iota(jnp.int32, (nb, nb), 1)
    tri8 = (c8 < r8).astype(jnp.float32)
    bloff = jnp.dot(tri8, bsum, preferred_element_type=jnp.float32)
    rank = jnp.concatenate(
        [ranks_local[b] + bloff[b:b + 1] for b in range(nb)], axis=0)

    counts = jnp.sum(sel, axis=0, keepdims=True)
    nblk_e = jnp.floor((counts + float(_BT - 1)) * (1.0 / _BT))
    up8 = (r8 < c8).astype(jnp.float32)
    blkstart = jnp.dot(nblk_e, up8, preferred_element_type=jnp.float32)
    pos = blkstart * float(_BT) + rank
    slot1 = jnp.sum(jnp.where(idx == a1, pos, 0.0), axis=1, keepdims=True)
    slot2 = jnp.sum(jnp.where(idx == a2, pos, 0.0), axis=1, keepdims=True)
    slots_ref[...] = jnp.concatenate([slot1, slot2], axis=1).astype(jnp.int32)

    tvec = lax.broadcasted_iota(jnp.int32, (1, _T), 1)
    tvec_hi = (tvec // 256).astype(jnp.float32)
    tvec_lo = (tvec % 256).astype(jnp.float32)
    sblk = 512
    tid_blocks = []
    for sb in range(_NPAD // sblk):
        srow = (lax.broadcasted_iota(jnp.int32, (1, sblk), 1)
                + sb * sblk).astype(jnp.float32)
        m = ((slot1 == srow) | (slot2 == srow)).astype(jnp.float32)
        hi = jnp.dot(tvec_hi, m, preferred_element_type=jnp.float32)
        lo = jnp.dot(tvec_lo, m, preferred_element_type=jnp.float32)
        tid_blocks.append(hi * 256.0 + lo)
    stid_ref[...] = jnp.concatenate(tid_blocks, axis=1).astype(jnp.int32)

    jlane = lax.broadcasted_iota(jnp.int32, (1, 32), 1).astype(jnp.float32)
    be = jnp.full((1, 32), -1.0, jnp.float32)
    for e in range(_E):
        be = be + (jlane >= jnp.broadcast_to(blkstart[:, e:e + 1], (1, 32))
                   ).astype(jnp.float32)
    nblk_total = jnp.sum(nblk_e, axis=1, keepdims=True)
    lane = lax.broadcasted_iota(jnp.int32, (1, 32), 1)
    meta_ref[...] = jnp.where(
        lane == 31, jnp.broadcast_to(nblk_total, (1, 32)), be).astype(jnp.int32)

    sl = jnp.dot(x, wsg_ref[...], preferred_element_type=jnp.float32)
    sgate_ref[...] = jax.nn.sigmoid(sl)


def _make_sc_dispatch_gather():
    info = plsc.get_sparse_core_info()
    nc, ns = info.num_cores, info.num_subcores
    nw = nc * ns
    per_w = _NPAD // nw
    chunk = 32
    nchunk = per_w // chunk
    mesh = plsc.VectorSubcoreMesh(core_axis_name="c", subcore_axis_name="s")

    @functools.partial(
        pl.kernel,
        out_type=jax.ShapeDtypeStruct((_NPAD, _D // 2), jnp.int32),
        mesh=mesh,
        scratch_types=[
            pltpu.VMEM((chunk,), jnp.int32),
            pltpu.VMEM((chunk,), jnp.int32),
            pltpu.VMEM((chunk, _D // 2), jnp.int32),
            pltpu.VMEM((chunk, _D // 2), jnp.int32),
            pltpu.SemaphoreType.DMA,
            pltpu.SemaphoreType.DMA,
        ],
    )
    def dispatch_k(stid_hbm, xb_hbm, out_hbm, idx0, idx1, rows0, rows1,
                   sem0, sem1):
        wid = lax.axis_index("s") * nc + lax.axis_index("c")
        base = wid * per_w
        idxs = (idx0, idx1)
        rows = (rows0, rows1)
        sems = (sem0, sem1)
        copies = [None, None]
        for c in range(nchunk):
            pltpu.sync_copy(stid_hbm.at[pl.ds(base + c * chunk, chunk)],
                            idxs[c % 2])
            copies[c % 2] = pltpu.async_copy(
                xb_hbm.at[idxs[c % 2]], rows[c % 2], sems[c % 2])
            if c > 0:
                copies[(c - 1) % 2].wait()
                pltpu.sync_copy(rows[(c - 1) % 2],
                                out_hbm.at[pl.ds(base + (c - 1) * chunk, chunk)])
        copies[(nchunk - 1) % 2].wait()
        pltpu.sync_copy(rows[(nchunk - 1) % 2],
                        out_hbm.at[pl.ds(base + (nchunk - 1) * chunk, chunk)])

    return dispatch_k


def _make_sc_gather_y():
    info = plsc.get_sparse_core_info()
    nc, ns = info.num_cores, info.num_subcores
    nw = nc * ns
    per_w = 2 * _T // nw
    chunk = 32
    nchunk = per_w // chunk
    mesh = plsc.VectorSubcoreMesh(core_axis_name="c", subcore_axis_name="s")

    @functools.partial(
        pl.kernel,
        out_type=jax.ShapeDtypeStruct((2 * _T, _D // 2), jnp.int32),
        mesh=mesh,
        scratch_types=[
            pltpu.VMEM((chunk,), jnp.int32),
            pltpu.VMEM((chunk,), jnp.int32),
            pltpu.VMEM((chunk, _D // 2), jnp.int32),
            pltpu.VMEM((chunk, _D // 2), jnp.int32),
            pltpu.SemaphoreType.DMA,
            pltpu.SemaphoreType.DMA,
        ],
    )
    def gather_k(slots_hbm, y_hbm, out_hbm, idx0, idx1, rows0, rows1,
                 sem0, sem1):
        wid = lax.axis_index("s") * nc + lax.axis_index("c")
        base = wid * per_w
        idxs = (idx0, idx1)
        rows = (rows0, rows1)
        sems = (sem0, sem1)
        copies = [None, None]
        for c in range(nchunk):
            pltpu.sync_copy(slots_hbm.at[pl.ds(base + c * chunk, chunk)],
                            idxs[c % 2])
            copies[c % 2] = pltpu.async_copy(
                y_hbm.at[idxs[c % 2]], rows[c % 2], sems[c % 2])
            if c > 0:
                copies[(c - 1) % 2].wait()
                pltpu.sync_copy(rows[(c - 1) % 2],
                                out_hbm.at[pl.ds(base + (c - 1) * chunk, chunk)])
        copies[(nchunk - 1) % 2].wait()
        pltpu.sync_copy(rows[(nchunk - 1) % 2],
                        out_hbm.at[pl.ds(base + (nchunk - 1) * chunk, chunk)])

    return gather_k


def _grouped_body(meta_ref, xs_ref, wg_ref, wu_ref, wd_ref, y_ref):
    b = pl.program_id(0)

    @pl.when(b < meta_ref[31])
    def _():
        xs = xs_ref[...]
        g = jnp.dot(xs, wg_ref[0], preferred_element_type=jnp.float32)
        u = jnp.dot(xs, wu_ref[0], preferred_element_type=jnp.float32)
        h = ((g * jax.nn.sigmoid(g)) * u).astype(jnp.bfloat16)
        y_ref[...] = jnp.dot(
            h, wd_ref[0], preferred_element_type=jnp.float32
        ).astype(jnp.bfloat16)


def _shared_body(xb_ref, wgu_ref, wd_ref, sgate_ref, out_ref):
    xb = xb_ref[...]
    gu = jnp.dot(xb, wgu_ref[...], preferred_element_type=jnp.float32)
    sg = gu[:, :_SFF]
    su = gu[:, _SFF:]
    hs = ((sg * jax.nn.sigmoid(sg)) * su).astype(jnp.bfloat16)
    sh = jnp.dot(hs, wd_ref[...], preferred_element_type=jnp.float32)
    out_ref[...] = sgate_ref[...] * sh


def _combine_body(sh_ref, yp_ref, w_ref, out_ref):
    w = w_ref[...]
    yp = yp_ref[...].astype(jnp.float32)
    out_ref[...] = (sh_ref[...]
                    + w[:, 0:1] * yp[:, :_D]
                    + w[:, 1:2] * yp[:, _D:])


_sc_kernel_cache = {}


def _get_sc_kernels():
    if "k" not in _sc_kernel_cache:
        _sc_kernel_cache["k"] = (_make_sc_dispatch_gather(),
                                 _make_sc_gather_y())
    return _sc_kernel_cache["k"]


def kernel(hidden_states, W_router, W_gate, W_up, W_down, Ws_gate_up, Ws_down,
           W_shared_gate):
    x = hidden_states
    xb = x.astype(jnp.bfloat16)
    wg = W_gate.astype(jnp.bfloat16)
    wu = W_up.astype(jnp.bfloat16)
    wd = W_down.astype(jnp.bfloat16)
    wsgu = Ws_gate_up.astype(jnp.bfloat16)
    wsd = Ws_down.astype(jnp.bfloat16)

    slots, topk_w, sgate, meta, stid = pl.pallas_call(
        _router_body,
        grid=(1,),
        in_specs=[
            pl.BlockSpec((_T, _D), lambda i: (0, 0)),
            pl.BlockSpec((_D, _E), lambda i: (0, 0)),
            pl.BlockSpec((_D, 1), lambda i: (0, 0)),
        ],
        out_specs=[
            pl.BlockSpec((_T, _K), lambda i: (0, 0)),
            pl.BlockSpec((_T, _K), lambda i: (0, 0)),
            pl.BlockSpec((_T, 1), lambda i: (0, 0)),
            pl.BlockSpec((1, 32), lambda i: (0, 0)),
            pl.BlockSpec((1, _NPAD), lambda i: (0, 0)),
        ],
        out_shape=[
            jax.ShapeDtypeStruct((_T, _K), jnp.int32),
            jax.ShapeDtypeStruct((_T, _K), jnp.float32),
            jax.ShapeDtypeStruct((_T, 1), jnp.float32),
            jax.ShapeDtypeStruct((1, 32), jnp.int32),
            jax.ShapeDtypeStruct((1, _NPAD), jnp.int32),
        ],
    )(x, W_router, W_shared_gate)

    slots_flat = slots.reshape(2 * _T)
    meta_flat = meta.reshape(32)
    stid_flat = stid.reshape(_NPAD)

    sc_dispatch_gather, sc_gather_y = _get_sc_kernels()
    xb_i32 = lax.bitcast_convert_type(
        xb.reshape(_T, _D // 2, 2), jnp.int32)
    x_sorted_i32 = sc_dispatch_gather(stid_flat, xb_i32)
    x_sorted = lax.bitcast_convert_type(
        x_sorted_i32, jnp.bfloat16).reshape(_NPAD, _D)

    y = pl.pallas_call(
        _grouped_body,
        grid_spec=pltpu.PrefetchScalarGridSpec(
            num_scalar_prefetch=1,
            grid=(_NBLK,),
            in_specs=[
                pl.BlockSpec((_BT, _D), lambda b, m: (b, 0)),
                pl.BlockSpec((1, _D, _FF), lambda b, m: (m[b], 0, 0)),
                pl.BlockSpec((1, _D, _FF), lambda b, m: (m[b], 0, 0)),
                pl.BlockSpec((1, _FF, _D), lambda b, m: (m[b], 0, 0)),
            ],
            out_specs=pl.BlockSpec((_BT, _D), lambda b, m: (b, 0)),
        ),
        out_shape=jax.ShapeDtypeStruct((_NPAD, _D), jnp.bfloat16),
        compiler_params=pltpu.CompilerParams(
            dimension_semantics=("arbitrary",)),
    )(meta_flat, x_sorted, wg, wu, wd)

    y_i32 = lax.bitcast_convert_type(y.reshape(_NPAD, _D // 2, 2), jnp.int32)
    ypair_i32 = sc_gather_y(slots_flat, y_i32)
    ypair2 = lax.bitcast_convert_type(
        ypair_i32, jnp.bfloat16).reshape(_T, 2 * _D)

    bs = 512
    shared = pl.pallas_call(
        _shared_body,
        grid=(_T // bs,),
        in_specs=[
            pl.BlockSpec((bs, _D), lambda t: (t, 0)),
            pl.BlockSpec((_D, 2 * _SFF), lambda t: (0, 0)),
            pl.BlockSpec((_SFF, _D), lambda t: (0, 0)),
            pl.BlockSpec((bs, 1), lambda t: (t, 0)),
        ],
        out_specs=pl.BlockSpec((bs, _D), lambda t: (t, 0)),
        out_shape=jax.ShapeDtypeStruct((_T, _D), jnp.float32),
    )(xb, wsgu, wsd, sgate)

    out = pl.pallas_call(
        _combine_body,
        grid=(_T // bs,),
        in_specs=[
            pl.BlockSpec((bs, _D), lambda t: (t, 0)),
            pl.BlockSpec((bs, 2 * _D), lambda t: (t, 0)),
            pl.BlockSpec((bs, _K), lambda t: (t, 0)),
        ],
        out_specs=pl.BlockSpec((bs, _D), lambda t: (t, 0)),
        out_shape=jax.ShapeDtypeStruct((_T, _D), jnp.float32),
    )(shared, ypair2, topk_w)
    return out


# speedup vs baseline: 6.9224x; 6.9224x over previous
import functools

import jax
import jax.numpy as jnp
from jax import lax
from jax.experimental import pallas as pl
from jax.experimental.pallas import tpu as pltpu
from jax.experimental.pallas import tpu_sc as plsc

_T, _D, _E, _K, _FF, _SFF = 2048, 2048, 8, 2, 768, 2048
_BT = 256
_NBLK = 24
_NPAD = _NBLK * _BT


def _router_body(x_ref, wr_ref, wsg_ref, slots_ref, w_ref, sgate_ref, meta_ref,
                 stid_ref):
    x = x_ref[...]
    logits = jnp.dot(x, wr_ref[...], preferred_element_type=jnp.float32)
    idx = lax.broadcasted_iota(jnp.int32, (_T, _E), 1)
    m1 = jnp.max(logits, axis=1, keepdims=True)
    a1 = jnp.min(jnp.where(logits == m1, idx, _E), axis=1, keepdims=True)
    masked = jnp.where(idx == a1, -jnp.inf, logits)
    m2 = jnp.max(masked, axis=1, keepdims=True)
    a2 = jnp.min(jnp.where(masked == m2, idx, _E), axis=1, keepdims=True)
    w1 = jax.nn.sigmoid(m1 - m2)
    w_ref[...] = jnp.concatenate([w1, 1.0 - w1], axis=1)

    sel = ((idx == a1) | (idx == a2)).astype(jnp.float32)

    r = lax.broadcasted_iota(jnp.int32, (_BT, _BT), 0)
    c = lax.broadcasted_iota(jnp.int32, (_BT, _BT), 1)
    tri = (c < r).astype(jnp.float32)
    nb = _T // _BT
    bsums = []
    ranks_local = []
    for b in range(nb):
        sb = sel[b * _BT:(b + 1) * _BT]
        ranks_local.append(jnp.dot(tri, sb, preferred_element_type=jnp.float32))
        bsums.append(jnp.sum(sb, axis=0, keepdims=True))
    bsum = jnp.concatenate(bsums, axis=0)
    r8 = lax.broadcasted_iota(jnp.int32, (nb, nb), 0)
    c8 = lax.broadcasted_iota(jnp.int32, (nb, nb), 1)
    tri8 = (c8 < r8).astype(jnp.float32)
    bloff = jnp.dot(tri8, bsum, preferred_element_type=jnp.float32)
    rank = jnp.concatenate(
        [ranks_local[b] + bloff[b:b + 1] for b in range(nb)], axis=0)

    counts = jnp.sum(sel, axis=0, keepdims=True)
    nblk_e = jnp.floor((counts + float(_BT - 1)) * (1.0 / _BT))
    up8 = (r8 < c8).astype(jnp.float32)
    blkstart = jnp.dot(nblk_e, up8, preferred_element_type=jnp.float32)
    pos = blkstart * float(_BT) + rank
    slot1 = jnp.sum(jnp.where(idx == a1, pos, 0.0), axis=1, keepdims=True)
    slot2 = jnp.sum(jnp.where(idx == a2, pos, 0.0), axis=1, keepdims=True)
    slots_ref[...] = jnp.concatenate([slot1, slot2], axis=1).astype(jnp.int32)

    tvec = lax.broadcasted_iota(jnp.int32, (1, _T), 1)
    tvec_hi = (tvec // 256).astype(jnp.float32)
    tvec_lo = (tvec % 256).astype(jnp.float32)
    sblk = 512
    tid_blocks = []
    for sb in range(_NPAD // sblk):
        srow = (lax.broadcasted_iota(jnp.int32, (1, sblk), 1)
                + sb * sblk).astype(jnp.float32)
        m = ((slot1 == srow) | (slot2 == srow)).astype(jnp.float32)
        hi = jnp.dot(tvec_hi, m, preferred_element_type=jnp.float32)
        lo = jnp.dot(tvec_lo, m, preferred_element_type=jnp.float32)
        tid_blocks.append(hi * 256.0 + lo)
    stid_ref[...] = jnp.concatenate(tid_blocks, axis=1).astype(jnp.int32)

    jlane = lax.broadcasted_iota(jnp.int32, (1, 32), 1).astype(jnp.float32)
    be = jnp.full((1, 32), -1.0, jnp.float32)
    for e in range(_E):
        be = be + (jlane >= jnp.broadcast_to(blkstart[:, e:e + 1], (1, 32))
                   ).astype(jnp.float32)
    nblk_total = jnp.sum(nblk_e, axis=1, keepdims=True)
    lane = lax.broadcasted_iota(jnp.int32, (1, 32), 1)
    meta_ref[...] = jnp.where(
        lane == 31, jnp.broadcast_to(nblk_total, (1, 32)), be).astype(jnp.int32)

    sl = jnp.dot(x, wsg_ref[...], preferred_element_type=jnp.float32)
    sgate_ref[...] = jax.nn.sigmoid(sl)


def _make_sc_dispatch_gather():
    info = plsc.get_sparse_core_info()
    nc, ns = info.num_cores, info.num_subcores
    nw = nc * ns
    per_w = _NPAD // nw
    chunk = 32
    nchunk = per_w // chunk
    mesh = plsc.VectorSubcoreMesh(core_axis_name="c", subcore_axis_name="s")

    @functools.partial(
        pl.kernel,
        out_type=jax.ShapeDtypeStruct((_NPAD, _D // 2), jnp.int32),
        mesh=mesh,
        scratch_types=[
            pltpu.VMEM((chunk,), jnp.int32),
            pltpu.VMEM((chunk,), jnp.int32),
            pltpu.VMEM((chunk, _D // 2), jnp.int32),
            pltpu.VMEM((chunk, _D // 2), jnp.int32),
            pltpu.SemaphoreType.DMA,
            pltpu.SemaphoreType.DMA,
        ],
    )
    def dispatch_k(stid_hbm, xb_hbm, out_hbm, idx0, idx1, rows0, rows1,
                   sem0, sem1):
        wid = lax.axis_index("s") * nc + lax.axis_index("c")
        base = wid * per_w
        idxs = (idx0, idx1)
        rows = (rows0, rows1)
        sems = (sem0, sem1)
        copies = [None, None]
        for c in range(nchunk):
            pltpu.sync_copy(stid_hbm.at[pl.ds(base + c * chunk, chunk)],
                            idxs[c % 2])
            copies[c % 2] = pltpu.async_copy(
                xb_hbm.at[idxs[c % 2]], rows[c % 2], sems[c % 2])
            if c > 0:
                copies[(c - 1) % 2].wait()
                pltpu.sync_copy(rows[(c - 1) % 2],
                                out_hbm.at[pl.ds(base + (c - 1) * chunk, chunk)])
        copies[(nchunk - 1) % 2].wait()
        pltpu.sync_copy(rows[(nchunk - 1) % 2],
                        out_hbm.at[pl.ds(base + (nchunk - 1) * chunk, chunk)])

    return dispatch_k


def _make_sc_gather_y():
    info = plsc.get_sparse_core_info()
    nc, ns = info.num_cores, info.num_subcores
    nw = nc * ns
    per_w = 2 * _T // nw
    chunk = 16
    nchunk = per_w // chunk
    mesh = plsc.VectorSubcoreMesh(core_axis_name="c", subcore_axis_name="s")

    @functools.partial(
        pl.kernel,
        out_type=jax.ShapeDtypeStruct((2 * _T, _D), jnp.float32),
        mesh=mesh,
        scratch_types=[
            pltpu.VMEM((chunk,), jnp.int32),
            pltpu.VMEM((chunk,), jnp.int32),
            pltpu.VMEM((chunk, _D), jnp.float32),
            pltpu.VMEM((chunk, _D), jnp.float32),
            pltpu.SemaphoreType.DMA,
            pltpu.SemaphoreType.DMA,
        ],
    )
    def gather_k(slots_hbm, y_hbm, out_hbm, idx0, idx1, rows0, rows1,
                 sem0, sem1):
        wid = lax.axis_index("s") * nc + lax.axis_index("c")
        base = wid * per_w
        idxs = (idx0, idx1)
        rows = (rows0, rows1)
        sems = (sem0, sem1)
        copies = [None, None]
        for c in range(nchunk):
            pltpu.sync_copy(slots_hbm.at[pl.ds(base + c * chunk, chunk)],
                            idxs[c % 2])
            copies[c % 2] = pltpu.async_copy(
                y_hbm.at[idxs[c % 2]], rows[c % 2], sems[c % 2])
            if c > 0:
                copies[(c - 1) % 2].wait()
                pltpu.sync_copy(rows[(c - 1) % 2],
                                out_hbm.at[pl.ds(base + (c - 1) * chunk, chunk)])
        copies[(nchunk - 1) % 2].wait()
        pltpu.sync_copy(rows[(nchunk - 1) % 2],
                        out_hbm.at[pl.ds(base + (nchunk - 1) * chunk, chunk)])

    return gather_k


def _grouped_body(meta_ref, xs_ref, wg_ref, wu_ref, wd_ref, y_ref):
    b = pl.program_id(0)

    @pl.when(b < meta_ref[31])
    def _():
        xs = xs_ref[...]
        g = jnp.dot(xs, wg_ref[0], preferred_element_type=jnp.float32)
        u = jnp.dot(xs, wu_ref[0], preferred_element_type=jnp.float32)
        h = ((g * jax.nn.sigmoid(g)) * u).astype(jnp.bfloat16)
        y_ref[...] = jnp.dot(h, wd_ref[0], preferred_element_type=jnp.float32)


def _shared_body(xb_ref, wgu_ref, wd_ref, sgate_ref, out_ref):
    xb = xb_ref[...]
    gu = jnp.dot(xb, wgu_ref[...], preferred_element_type=jnp.float32)
    sg = gu[:, :_SFF]
    su = gu[:, _SFF:]
    hs = ((sg * jax.nn.sigmoid(sg)) * su).astype(jnp.bfloat16)
    sh = jnp.dot(hs, wd_ref[...], preferred_element_type=jnp.float32)
    out_ref[...] = sgate_ref[...] * sh


def _combine_body(sh_ref, yp_ref, w_ref, out_ref):
    w = w_ref[...]
    yp = yp_ref[...]
    out_ref[...] = (sh_ref[...]
                    + w[:, 0:1] * yp[:, :_D]
                    + w[:, 1:2] * yp[:, _D:])


_sc_kernel_cache = {}


def _get_sc_kernels():
    if "k" not in _sc_kernel_cache:
        _sc_kernel_cache["k"] = (_make_sc_dispatch_gather(),
                                 _make_sc_gather_y())
    return _sc_kernel_cache["k"]


def kernel(hidden_states, W_router, W_gate, W_up, W_down, Ws_gate_up, Ws_down,
           W_shared_gate):
    x = hidden_states
    xb = x.astype(jnp.bfloat16)
    wg = W_gate.astype(jnp.bfloat16)
    wu = W_up.astype(jnp.bfloat16)
    wd = W_down.astype(jnp.bfloat16)
    wsgu = Ws_gate_up.astype(jnp.bfloat16)
    wsd = Ws_down.astype(jnp.bfloat16)

    slots, topk_w, sgate, meta, stid = pl.pallas_call(
        _router_body,
        grid=(1,),
        in_specs=[
            pl.BlockSpec((_T, _D), lambda i: (0, 0)),
            pl.BlockSpec((_D, _E), lambda i: (0, 0)),
            pl.BlockSpec((_D, 1), lambda i: (0, 0)),
        ],
        out_specs=[
            pl.BlockSpec((_T, _K), lambda i: (0, 0)),
            pl.BlockSpec((_T, _K), lambda i: (0, 0)),
            pl.BlockSpec((_T, 1), lambda i: (0, 0)),
            pl.BlockSpec((1, 32), lambda i: (0, 0)),
            pl.BlockSpec((1, _NPAD), lambda i: (0, 0)),
        ],
        out_shape=[
            jax.ShapeDtypeStruct((_T, _K), jnp.int32),
            jax.ShapeDtypeStruct((_T, _K), jnp.float32),
            jax.ShapeDtypeStruct((_T, 1), jnp.float32),
            jax.ShapeDtypeStruct((1, 32), jnp.int32),
            jax.ShapeDtypeStruct((1, _NPAD), jnp.int32),
        ],
    )(x, W_router, W_shared_gate)

    slots_flat = slots.reshape(2 * _T)
    meta_flat = meta.reshape(32)
    stid_flat = stid.reshape(_NPAD)

    sc_dispatch_gather, sc_gather_y = _get_sc_kernels()
    xb_i32 = lax.bitcast_convert_type(
        xb.reshape(_T, _D // 2, 2), jnp.int32)
    x_sorted_i32 = sc_dispatch_gather(stid_flat, xb_i32)
    x_sorted = lax.bitcast_convert_type(
        x_sorted_i32, jnp.bfloat16).reshape(_NPAD, _D)

    y = pl.pallas_call(
        _grouped_body,
        grid_spec=pltpu.PrefetchScalarGridSpec(
            num_scalar_prefetch=1,
            grid=(_NBLK,),
            in_specs=[
                pl.BlockSpec((_BT, _D), lambda b, m: (b, 0)),
                pl.BlockSpec((1, _D, _FF), lambda b, m: (m[b], 0, 0)),
                pl.BlockSpec((1, _D, _FF), lambda b, m: (m[b], 0, 0)),
                pl.BlockSpec((1, _FF, _D), lambda b, m: (m[b], 0, 0)),
            ],
            out_specs=pl.BlockSpec((_BT, _D), lambda b, m: (b, 0)),
        ),
        out_shape=jax.ShapeDtypeStruct((_NPAD, _D), jnp.float32),
        compiler_params=pltpu.CompilerParams(
            dimension_semantics=("arbitrary",)),
    )(meta_flat, x_sorted, wg, wu, wd)

    ypair = sc_gather_y(slots_flat, y)
    ypair2 = ypair.reshape(_T, 2 * _D)

    bs = 512
    shared = pl.pallas_call(
        _shared_body,
        grid=(_T // bs,),
        in_specs=[
            pl.BlockSpec((bs, _D), lambda t: (t, 0)),
            pl.BlockSpec((_D, 2 * _SFF), lambda t: (0, 0)),
            pl.BlockSpec((_SFF, _D), lambda t: (0, 0)),
            pl.BlockSpec((bs, 1), lambda t: (t, 0)),
        ],
        out_specs=pl.BlockSpec((bs, _D), lambda t: (t, 0)),
        out_shape=jax.ShapeDtypeStruct((_T, _D), jnp.float32),
    )(xb, wsgu, wsd, sgate)

    out = pl.pallas_call(
        _combine_body,
        grid=(_T // bs,),
        in_specs=[
            pl.BlockSpec((bs, _D), lambda t: (t, 0)),
            pl.BlockSpec((bs, 2 * _D), lambda t: (t, 0)),
            pl.BlockSpec((bs, _K), lambda t: (t, 0)),
        ],
        out_specs=pl.BlockSpec((bs, _D), lambda t: (t, 0)),
        out_shape=jax.ShapeDtypeStruct((_T, _D), jnp.float32),
    )(shared, ypair2, topk_w)
    return out


# speedup vs baseline: 15.8630x; 2.2915x over previous
import jax
import jax.numpy as jnp
from jax.experimental import pallas as pl
from jax.experimental.pallas import tpu as pltpu

_T, _D, _E, _K, _FF, _SFF = 2048, 2048, 8, 2, 768, 2048


def _router_body(x_ref, wr_ref, wsg_ref, comb_ref, sgate_ref):
    x = x_ref[...]
    logits = jnp.dot(x, wr_ref[...], preferred_element_type=jnp.float32)
    idx = jax.lax.broadcasted_iota(jnp.int32, logits.shape, 1)
    m1 = jnp.max(logits, axis=1, keepdims=True)
    a1 = jnp.min(jnp.where(logits == m1, idx, _E), axis=1, keepdims=True)
    masked = jnp.where(idx == a1, -jnp.inf, logits)
    m2 = jnp.max(masked, axis=1, keepdims=True)
    a2 = jnp.min(jnp.where(masked == m2, idx, _E), axis=1, keepdims=True)
    w1 = jax.nn.sigmoid(m1 - m2)
    w2 = 1.0 - w1
    comb_ref[...] = jnp.where(idx == a1, w1, 0.0) + jnp.where(idx == a2, w2, 0.0)
    sl = jnp.dot(x, wsg_ref[...], preferred_element_type=jnp.float32)
    sgate_ref[...] = jax.nn.sigmoid(sl)


def _routed_body(comb_ref, xb_ref, wg_ref, wu_ref, wd_ref, acc_ref):
    e = pl.program_id(1)
    xb = xb_ref[...]
    g = jnp.dot(xb, wg_ref[0], preferred_element_type=jnp.float32)
    u = jnp.dot(xb, wu_ref[0], preferred_element_type=jnp.float32)
    h = (g * jax.nn.sigmoid(g)) * u
    comb = comb_ref[...]
    idx = jax.lax.broadcasted_iota(jnp.int32, comb.shape, 1)
    w = jnp.sum(jnp.where(idx == e, comb, 0.0), axis=1, keepdims=True)
    hw = (h * w).astype(jnp.bfloat16)
    contrib = jnp.dot(hw, wd_ref[0], preferred_element_type=jnp.float32)

    @pl.when(e == 0)
    def _():
        acc_ref[...] = contrib

    @pl.when(e != 0)
    def _():
        acc_ref[...] += contrib


def _shared_body(xb_ref, wgu_ref, wd_ref, routed_ref, sgate_ref, out_ref):
    xb = xb_ref[...]
    gu = jnp.dot(xb, wgu_ref[...], preferred_element_type=jnp.float32)
    sg = gu[:, :_SFF]
    su = gu[:, _SFF:]
    hs = ((sg * jax.nn.sigmoid(sg)) * su).astype(jnp.bfloat16)
    sh = jnp.dot(hs, wd_ref[...], preferred_element_type=jnp.float32)
    out_ref[...] = routed_ref[...] + sgate_ref[...] * sh


def kernel(hidden_states, W_router, W_gate, W_up, W_down, Ws_gate_up, Ws_down,
           W_shared_gate):
    x = hidden_states
    xb = x.astype(jnp.bfloat16)
    wg = W_gate.astype(jnp.bfloat16)
    wu = W_up.astype(jnp.bfloat16)
    wd = W_down.astype(jnp.bfloat16)
    wsgu = Ws_gate_up.astype(jnp.bfloat16)
    wsd = Ws_down.astype(jnp.bfloat16)

    comb, sgate = pl.pallas_call(
        _router_body,
        grid=(1,),
        in_specs=[
            pl.BlockSpec((_T, _D), lambda i: (0, 0)),
            pl.BlockSpec((_D, _E), lambda i: (0, 0)),
            pl.BlockSpec((_D, 1), lambda i: (0, 0)),
        ],
        out_specs=[
            pl.BlockSpec((_T, _E), lambda i: (0, 0)),
            pl.BlockSpec((_T, 1), lambda i: (0, 0)),
        ],
        out_shape=[
            jax.ShapeDtypeStruct((_T, _E), jnp.float32),
            jax.ShapeDtypeStruct((_T, 1), jnp.float32),
        ],
    )(x, W_router, W_shared_gate)

    bt = 1024
    routed = pl.pallas_call(
        _routed_body,
        grid=(_T // bt, _E),
        in_specs=[
            pl.BlockSpec((bt, _E), lambda t, e: (t, 0)),
            pl.BlockSpec((bt, _D), lambda t, e: (t, 0)),
            pl.BlockSpec((1, _D, _FF), lambda t, e: (e, 0, 0)),
            pl.BlockSpec((1, _D, _FF), lambda t, e: (e, 0, 0)),
            pl.BlockSpec((1, _FF, _D), lambda t, e: (e, 0, 0)),
        ],
        out_specs=pl.BlockSpec((bt, _D), lambda t, e: (t, 0)),
        out_shape=jax.ShapeDtypeStruct((_T, _D), jnp.float32),
        compiler_params=pltpu.CompilerParams(
            dimension_semantics=("arbitrary", "arbitrary")),
    )(comb, xb, wg, wu, wd)

    bs = 512
    out = pl.pallas_call(
        _shared_body,
        grid=(_T // bs,),
        in_specs=[
            pl.BlockSpec((bs, _D), lambda t: (t, 0)),
            pl.BlockSpec((_D, 2 * _SFF), lambda t: (0, 0)),
            pl.BlockSpec((_SFF, _D), lambda t: (0, 0)),
            pl.BlockSpec((bs, _D), lambda t: (t, 0)),
            pl.BlockSpec((bs, 1), lambda t: (t, 0)),
        ],
        out_specs=pl.BlockSpec((bs, _D), lambda t: (t, 0)),
        out_shape=jax.ShapeDtypeStruct((_T, _D), jnp.float32),
    )(xb, wsgu, wsd, routed, sgate)
    return out
